# baseline (speedup 1.0000x reference)
import functools

import jax
import jax.numpy as jnp
from jax import lax
from jax.experimental import pallas as pl
from jax.experimental.pallas import tpu as pltpu
from jax.experimental.pallas import tpu_sc as plsc

F = 128
N_NODES = 10000
N_EDGES = 320000
NCHUNK = 4
ECH = N_EDGES // NCHUNK

NC = 2
NS = 16
NW = NC * NS

IB = 128
KI = 8
EB = IB * KI
HB = EB // 2
NBH = ECH // EB
KMAXH = -(-NBH // NW)
TAIL_ROWS = (ECH - NBH * EB) // IB
TAIL = TAIL_ROWS * IB

BE = 2000
NEBH = ECH // BE

_BN_EPS = 1e-5


def _sc_mesh():
    return plsc.VectorSubcoreMesh(core_axis_name="c", subcore_axis_name="s",
                                  num_cores=NC, num_subcores=NS)


def _sc_gather(x, dst2, src2):
    @functools.partial(
        pl.kernel,
        out_type=(jax.ShapeDtypeStruct((ECH, F), jnp.float32),
                  jax.ShapeDtypeStruct((ECH, F), jnp.float32)),
        mesh=_sc_mesh(),
        scratch_types=[
            pltpu.VMEM((KI, IB), jnp.int32),
            pltpu.VMEM((2 * IB, F), jnp.float32),
            pltpu.VMEM((2 * IB, F), jnp.float32),
            pltpu.SemaphoreType.DMA,
            pltpu.SemaphoreType.DMA,
            pltpu.SemaphoreType.DMA,
            pltpu.SemaphoreType.DMA,
        ],
    )
    def k(x_hbm, dst_hbm, src_hbm, gi_hbm, gj_hbm, idx_v, r0, r1,
          g0, g1, w0, w1):
        wid = lax.axis_index("s") * NC + lax.axis_index("c")
        rows = (r0, r1)
        gsem = (g0, g1)
        wsem = (w0, w1)

        def one_phase(b, idx_hbm, out_hbm, njobs=KI // 2):
            pltpu.sync_copy(idx_hbm.at[pl.ds(b * KI, KI)], idx_v)

            def fire(t):
                i = t % 2
                return [
                    pltpu.async_copy(x_hbm.at[idx_v.at[2 * t + r]],
                                     rows[i].at[pl.ds(r * IB, IB)], gsem[i])
                    for r in range(2)
                ]

            gp = [None] * njobs
            gp[0] = fire(0)
            if njobs > 1:
                gp[1] = fire(1)
            for t in range(njobs):
                i = t % 2
                for cp in gp[t]:
                    cp.wait()
                wp = pltpu.async_copy(
                    rows[i], out_hbm.at[pl.ds(b * EB + t * 2 * IB, 2 * IB)],
                    wsem[i])
                wp.wait()
                if t + 2 < njobs:
                    gp[t + 2] = fire(t + 2)

        def body(kk, _):
            b = kk * NW + wid

            @pl.when(b < NBH)
            def _():
                one_phase(b, dst_hbm, gi_hbm)
                one_phase(b, src_hbm, gj_hbm)
            return 0

        lax.fori_loop(0, KMAXH, body, 0)

        @pl.when(wid == NW - 1)
        def _():
            for idx_hbm, out_hbm in ((dst_hbm, gi_hbm), (src_hbm, gj_hbm)):
                pltpu.sync_copy(idx_hbm.at[pl.ds(NBH * KI, TAIL_ROWS)],
                                idx_v.at[pl.ds(0, TAIL_ROWS)])
                cps = [
                    pltpu.async_copy(x_hbm.at[idx_v.at[r]],
                                     r0.at[pl.ds(r * IB, IB)], g0)
                    for r in range(TAIL_ROWS)
                ]
                for cp in cps:
                    cp.wait()
                pltpu.sync_copy(r0.at[pl.ds(0, TAIL)],
                                out_hbm.at[pl.ds(NBH * EB, TAIL)])

    return k(x, dst2, src2)


def _edge_mlp(gi, gj, ea, wc1, bc1, w2f, b2f, w2, b2, half):
    def body(gi_ref, gj_ref, ea_ref, wc1_ref, bc1_ref, w2f_ref, b2f_ref,
             w2_ref, b2_ref, hf_ref, h_ref, mom_ref):
        z = jnp.concatenate(
            [gi_ref[...].astype(jnp.bfloat16),
             gj_ref[...].astype(jnp.bfloat16),
             ea_ref[...].astype(jnp.bfloat16)], axis=1)
        u = jnp.dot(z, wc1_ref[...].astype(jnp.bfloat16),
                    preferred_element_type=jnp.float32)
        u = u + bc1_ref[...]
        s = (u * jax.nn.sigmoid(u)).astype(jnp.bfloat16)
        hf = jnp.dot(s[:, :F], w2f_ref[...].astype(jnp.bfloat16),
                     preferred_element_type=jnp.float32) + b2f_ref[...]
        h = jnp.dot(s[:, F:], w2_ref[...].astype(jnp.bfloat16),
                    preferred_element_type=jnp.float32) + b2_ref[...]
        hf_ref[...] = hf.astype(jnp.bfloat16)
        h_ref[...] = h.astype(jnp.bfloat16)

        @pl.when(pl.program_id(0) == 0)
        def _():
            mom_ref[...] = jnp.zeros_like(mom_ref)

        pad = jnp.zeros((6, F), jnp.float32)
        mom_ref[...] += jnp.concatenate(
            [jnp.sum(hf, axis=0, keepdims=True),
             jnp.sum(hf * hf, axis=0, keepdims=True), pad], axis=0)

    eb = lambda i: (i, 0)
    ebo = lambda i: (i + half * NEBH, 0)
    c0 = lambda i: (0, 0)
    return pl.pallas_call(
        body,
        grid=(NEBH,),
        in_specs=[
            pl.BlockSpec((BE, F), eb),
            pl.BlockSpec((BE, F), eb),
            pl.BlockSpec((BE, F), ebo),
            pl.BlockSpec((3 * F, 2 * F), c0),
            pl.BlockSpec((1, 2 * F), c0),
            pl.BlockSpec((F, F), c0),
            pl.BlockSpec((1, F), c0),
            pl.BlockSpec((F, F), c0),
            pl.BlockSpec((1, F), c0),
        ],
        out_specs=[pl.BlockSpec((BE, F), eb),
                   pl.BlockSpec((BE, F), eb),
                   pl.BlockSpec((8, F), c0)],
        out_shape=[jax.ShapeDtypeStruct((ECH, F), jnp.bfloat16),
                   jax.ShapeDtypeStruct((ECH, F), jnp.bfloat16),
                   jax.ShapeDtypeStruct((8, F), jnp.float32)],
    )(gi, gj, ea, wc1, bc1, w2f, b2f, w2, b2)


def _message(hf, h, moms, g_bni, b_bni):
    inv_e = 1.0 / N_EDGES

    def body(hf_ref, h_ref, m0_ref, m1_ref, m2_ref, m3_ref, g_ref, b_ref,
             msg_ref):
        mom = m0_ref[...] + m1_ref[...] + m2_ref[...] + m3_ref[...]
        mean = mom[0:1, :] * inv_e
        var = mom[1:2, :] * inv_e - mean * mean
        scale = g_ref[...] * lax.rsqrt(var + _BN_EPS)
        shift = b_ref[...] - mean * scale
        score = jax.nn.sigmoid(hf_ref[...].astype(jnp.float32) * scale + shift)
        msg_ref[...] = score * h_ref[...].astype(jnp.float32)

    eb = lambda i: (i, 0)
    c0 = lambda i: (0, 0)
    return pl.pallas_call(
        body,
        grid=(NEBH,),
        in_specs=[
            pl.BlockSpec((BE, F), eb),
            pl.BlockSpec((BE, F), eb),
            pl.BlockSpec((8, F), c0),
            pl.BlockSpec((8, F), c0),
            pl.BlockSpec((8, F), c0),
            pl.BlockSpec((8, F), c0),
            pl.BlockSpec((1, F), c0),
            pl.BlockSpec((1, F), c0),
        ],
        out_specs=pl.BlockSpec((BE, F), eb),
        out_shape=jax.ShapeDtypeStruct((ECH, F), jnp.float32),
    )(hf, h, moms[0], moms[1], moms[2], moms[3], g_bni, b_bni)


def _sc_scatter(msg, dst2, zeros):
    @functools.partial(
        pl.kernel,
        out_type=jax.ShapeDtypeStruct((NC, N_NODES, F), jnp.float32),
        mesh=_sc_mesh(),
        scratch_types=[
            pltpu.VMEM((KI, IB), jnp.int32),
            pltpu.VMEM((IB, F), jnp.float32),
            pltpu.VMEM((IB, F), jnp.float32),
            pltpu.SemaphoreType.DMA,
            pltpu.SemaphoreType.DMA,
            pltpu.VMEM_SHARED((N_NODES, F), jnp.float32),
        ],
    )
    def k(msg_hbm, dst_hbm, z_hbm, out_hbm, idx_v, m0, m1, s0, s1, acc):
        c = lax.axis_index("c")
        s = lax.axis_index("s")
        wid = s * NC + c
        mv = (m0, m1)
        msem = (s0, s1)

        @pl.when(s == 0)
        def _():
            pltpu.sync_copy(z_hbm, acc)

        plsc.subcore_barrier()

        def one_phase(e0, njobs):
            def fire(t):
                i = t % 2
                return pltpu.async_copy(
                    msg_hbm.at[pl.ds(e0 + t * IB, IB)], mv[i], msem[i])

            cp = [None] * njobs
            cp[0] = fire(0)
            if njobs > 1:
                cp[1] = fire(1)
            for t in range(njobs):
                cp[t].wait()
                pltpu.sync_copy(mv[t % 2], acc.at[idx_v.at[t]], add=True)
                if t + 2 < njobs:
                    cp[t + 2] = fire(t + 2)

        def body(kk, _):
            b = kk * NW + wid

            @pl.when(b < NBH)
            def _():
                pltpu.sync_copy(dst_hbm.at[pl.ds(b * KI, KI)], idx_v)
                one_phase(b * EB, KI)
            return 0

        lax.fori_loop(0, KMAXH, body, 0)

        @pl.when(wid == NW - 1)
        def _():
            pltpu.sync_copy(dst_hbm.at[pl.ds(NBH * KI, TAIL_ROWS)],
                            idx_v.at[pl.ds(0, TAIL_ROWS)])
            one_phase(NBH * EB, TAIL_ROWS)

        plsc.subcore_barrier()

        @pl.when(s == 0)
        def _():
            pltpu.sync_copy(acc, out_hbm.at[c])

    return k(msg, dst2, zeros)


def _final(parts, x, g_bn, b_bn):
    inv_n = 1.0 / N_NODES

    def body(p0_ref, p1_ref, p2_ref, p3_ref, x_ref, g_ref, b_ref, o_ref):
        out = (p0_ref[0] + p0_ref[1] + p1_ref[0] + p1_ref[1]
               + p2_ref[0] + p2_ref[1] + p3_ref[0] + p3_ref[1])
        mean = jnp.sum(out, axis=0, keepdims=True) * inv_n
        d = out - mean
        var = jnp.sum(d * d, axis=0, keepdims=True) * inv_n
        scale = g_ref[...] * lax.rsqrt(var + _BN_EPS)
        o_ref[...] = jnp.maximum(x_ref[...] + d * scale + b_ref[...], 0.0)

    return pl.pallas_call(
        body,
        out_shape=jax.ShapeDtypeStruct((N_NODES, F), jnp.float32),
    )(parts[0], parts[1], parts[2], parts[3], x, g_bn, b_bn)


def kernel(x, edge_index, edge_attr, W1f, b1f, W2f, b2f, W1, b1, W2, b2,
           g_bn, b_bn, g_bni, b_bni):
    src = edge_index[0]
    dst = edge_index[1]
    dst2 = [dst[q * ECH:(q + 1) * ECH].reshape(ECH // IB, IB)
            for q in range(NCHUNK)]
    src2 = [src[q * ECH:(q + 1) * ECH].reshape(ECH // IB, IB)
            for q in range(NCHUNK)]

    wc1 = jnp.concatenate([W1f, W1], axis=1)
    bc1 = jnp.concatenate([b1f, b1]).reshape(1, 2 * F)
    zeros = jnp.zeros((N_NODES, F), jnp.float32)

    g = [_sc_gather(x, dst2[q], src2[q]) for q in range(NCHUNK)]
    mlp = [_edge_mlp(g[q][0], g[q][1], edge_attr, wc1, bc1,
                     W2f, b2f.reshape(1, F), W2, b2.reshape(1, F), half=q)
           for q in range(NCHUNK)]
    moms = [m[2] for m in mlp]
    g1 = g_bni.reshape(1, F)
    b1r = b_bni.reshape(1, F)
    msgs = [_message(mlp[q][0], mlp[q][1], moms, g1, b1r)
            for q in range(NCHUNK)]
    parts = [_sc_scatter(msgs[q], dst2[q], zeros) for q in range(NCHUNK)]
    return _final(parts, x, g_bn.reshape(1, F), b_bn.reshape(1, F))


# speedup vs baseline: 1.0047x; 1.0047x over previous
import functools

import jax
import jax.numpy as jnp
from jax import lax
from jax.experimental import pallas as pl
from jax.experimental.pallas import tpu as pltpu
from jax.experimental.pallas import tpu_sc as plsc

F = 128
N_NODES = 10000
N_EDGES = 320000
NCHUNK = 4
ECH = N_EDGES // NCHUNK

NC = 2
NS = 16
NW = NC * NS

IB = 128
KI = 8
EB = IB * KI
HB = EB // 2
NBH = ECH // EB
KMAXH = -(-NBH // NW)
TAIL_ROWS = (ECH - NBH * EB) // IB
TAIL = TAIL_ROWS * IB

BE = 2000
NEBH = ECH // BE

_BN_EPS = 1e-5


def _sc_mesh():
    return plsc.VectorSubcoreMesh(core_axis_name="c", subcore_axis_name="s",
                                  num_cores=NC, num_subcores=NS)


def _sc_gather(x, dst2, src2):
    @functools.partial(
        pl.kernel,
        out_type=(jax.ShapeDtypeStruct((ECH, F), jnp.float32),
                  jax.ShapeDtypeStruct((ECH, F), jnp.float32)),
        mesh=_sc_mesh(),
        scratch_types=[
            pltpu.VMEM((KI, IB), jnp.int32),
            pltpu.VMEM((2 * IB, F), jnp.float32),
            pltpu.VMEM((2 * IB, F), jnp.float32),
            pltpu.VMEM((2 * IB, F), jnp.float32),
            pltpu.SemaphoreType.DMA,
            pltpu.SemaphoreType.DMA,
            pltpu.SemaphoreType.DMA,
            pltpu.SemaphoreType.DMA,
            pltpu.SemaphoreType.DMA,
            pltpu.SemaphoreType.DMA,
        ],
    )
    def k(x_hbm, dst_hbm, src_hbm, gi_hbm, gj_hbm, idx_v, r0, r1, r2,
          g0, g1, g2, w0, w1, w2):
        wid = lax.axis_index("s") * NC + lax.axis_index("c")
        rows = (r0, r1, r2)
        gsem = (g0, g1, g2)
        wsem = (w0, w1, w2)

        def one_phase(b, idx_hbm, out_hbm, njobs=KI // 2):
            pltpu.sync_copy(idx_hbm.at[pl.ds(b * KI, KI)], idx_v)

            def fire(t):
                i = t % 3
                return [
                    pltpu.async_copy(x_hbm.at[idx_v.at[2 * t + r]],
                                     rows[i].at[pl.ds(r * IB, IB)], gsem[i])
                    for r in range(2)
                ]

            gp = [None] * njobs
            wp = [None] * njobs
            for t in range(min(3, njobs)):
                gp[t] = fire(t)
            for t in range(njobs):
                i = t % 3
                for cp in gp[t]:
                    cp.wait()
                wp[t] = pltpu.async_copy(
                    rows[i], out_hbm.at[pl.ds(b * EB + t * 2 * IB, 2 * IB)],
                    wsem[i])
                if t + 3 < njobs:
                    wp[t].wait()
                    gp[t + 3] = fire(t + 3)
            for t in range(max(0, njobs - 3), njobs):
                wp[t].wait()

        def body(kk, _):
            b = kk * NW + wid

            @pl.when(b < NBH)
            def _():
                one_phase(b, dst_hbm, gi_hbm)
                one_phase(b, src_hbm, gj_hbm)
            return 0

        lax.fori_loop(0, KMAXH, body, 0)

        @pl.when(wid == NW - 1)
        def _():
            for idx_hbm, out_hbm in ((dst_hbm, gi_hbm), (src_hbm, gj_hbm)):
                pltpu.sync_copy(idx_hbm.at[pl.ds(NBH * KI, TAIL_ROWS)],
                                idx_v.at[pl.ds(0, TAIL_ROWS)])
                cps = [
                    pltpu.async_copy(x_hbm.at[idx_v.at[r]],
                                     r0.at[pl.ds(r * IB, IB)], g0)
                    for r in range(TAIL_ROWS)
                ]
                for cp in cps:
                    cp.wait()
                pltpu.sync_copy(r0.at[pl.ds(0, TAIL)],
                                out_hbm.at[pl.ds(NBH * EB, TAIL)])

    return k(x, dst2, src2)


def _edge_mlp(gi, gj, ea, wc1, bc1, w2f, b2f, w2, b2, half):
    def body(gi_ref, gj_ref, ea_ref, wc1_ref, bc1_ref, w2f_ref, b2f_ref,
             w2_ref, b2_ref, hf_ref, h_ref, mom_ref):
        z = jnp.concatenate(
            [gi_ref[...].astype(jnp.bfloat16),
             gj_ref[...].astype(jnp.bfloat16),
             ea_ref[...].astype(jnp.bfloat16)], axis=1)
        u = jnp.dot(z, wc1_ref[...].astype(jnp.bfloat16),
                    preferred_element_type=jnp.float32)
        u = u + bc1_ref[...]
        s = (u * jax.nn.sigmoid(u)).astype(jnp.bfloat16)
        hf = jnp.dot(s[:, :F], w2f_ref[...].astype(jnp.bfloat16),
                     preferred_element_type=jnp.float32) + b2f_ref[...]
        h = jnp.dot(s[:, F:], w2_ref[...].astype(jnp.bfloat16),
                    preferred_element_type=jnp.float32) + b2_ref[...]
        hf_ref[...] = hf.astype(jnp.bfloat16)
        h_ref[...] = h.astype(jnp.bfloat16)

        @pl.when(pl.program_id(0) == 0)
        def _():
            mom_ref[...] = jnp.zeros_like(mom_ref)

        pad = jnp.zeros((6, F), jnp.float32)
        mom_ref[...] += jnp.concatenate(
            [jnp.sum(hf, axis=0, keepdims=True),
             jnp.sum(hf * hf, axis=0, keepdims=True), pad], axis=0)

    eb = lambda i: (i, 0)
    ebo = lambda i: (i + half * NEBH, 0)
    c0 = lambda i: (0, 0)
    return pl.pallas_call(
        body,
        grid=(NEBH,),
        in_specs=[
            pl.BlockSpec((BE, F), eb),
            pl.BlockSpec((BE, F), eb),
            pl.BlockSpec((BE, F), ebo),
            pl.BlockSpec((3 * F, 2 * F), c0),
            pl.BlockSpec((1, 2 * F), c0),
            pl.BlockSpec((F, F), c0),
            pl.BlockSpec((1, F), c0),
            pl.BlockSpec((F, F), c0),
            pl.BlockSpec((1, F), c0),
        ],
        out_specs=[pl.BlockSpec((BE, F), eb),
                   pl.BlockSpec((BE, F), eb),
                   pl.BlockSpec((8, F), c0)],
        out_shape=[jax.ShapeDtypeStruct((ECH, F), jnp.bfloat16),
                   jax.ShapeDtypeStruct((ECH, F), jnp.bfloat16),
                   jax.ShapeDtypeStruct((8, F), jnp.float32)],
    )(gi, gj, ea, wc1, bc1, w2f, b2f, w2, b2)


def _message(hf, h, moms, g_bni, b_bni):
    inv_e = 1.0 / N_EDGES

    def body(hf_ref, h_ref, m0_ref, m1_ref, m2_ref, m3_ref, g_ref, b_ref,
             msg_ref):
        mom = m0_ref[...] + m1_ref[...] + m2_ref[...] + m3_ref[...]
        mean = mom[0:1, :] * inv_e
        var = mom[1:2, :] * inv_e - mean * mean
        scale = g_ref[...] * lax.rsqrt(var + _BN_EPS)
        shift = b_ref[...] - mean * scale
        score = jax.nn.sigmoid(hf_ref[...].astype(jnp.float32) * scale + shift)
        msg_ref[...] = score * h_ref[...].astype(jnp.float32)

    eb = lambda i: (i, 0)
    c0 = lambda i: (0, 0)
    return pl.pallas_call(
        body,
        grid=(NEBH,),
        in_specs=[
            pl.BlockSpec((BE, F), eb),
            pl.BlockSpec((BE, F), eb),
            pl.BlockSpec((8, F), c0),
            pl.BlockSpec((8, F), c0),
            pl.BlockSpec((8, F), c0),
            pl.BlockSpec((8, F), c0),
            pl.BlockSpec((1, F), c0),
            pl.BlockSpec((1, F), c0),
        ],
        out_specs=pl.BlockSpec((BE, F), eb),
        out_shape=jax.ShapeDtypeStruct((ECH, F), jnp.float32),
    )(hf, h, moms[0], moms[1], moms[2], moms[3], g_bni, b_bni)


def _sc_scatter(msg, dst2, zeros):
    @functools.partial(
        pl.kernel,
        out_type=jax.ShapeDtypeStruct((NC, N_NODES, F), jnp.float32),
        mesh=_sc_mesh(),
        scratch_types=[
            pltpu.VMEM((KI, IB), jnp.int32),
            pltpu.VMEM((IB, F), jnp.float32),
            pltpu.VMEM((IB, F), jnp.float32),
            pltpu.SemaphoreType.DMA,
            pltpu.SemaphoreType.DMA,
            pltpu.VMEM_SHARED((N_NODES, F), jnp.float32),
        ],
    )
    def k(msg_hbm, dst_hbm, z_hbm, out_hbm, idx_v, m0, m1, s0, s1, acc):
        c = lax.axis_index("c")
        s = lax.axis_index("s")
        wid = s * NC + c
        mv = (m0, m1)
        msem = (s0, s1)

        @pl.when(s == 0)
        def _():
            pltpu.sync_copy(z_hbm, acc)

        plsc.subcore_barrier()

        def one_phase(e0, njobs):
            def fire(t):
                i = t % 2
                return pltpu.async_copy(
                    msg_hbm.at[pl.ds(e0 + t * IB, IB)], mv[i], msem[i])

            cp = [None] * njobs
            cp[0] = fire(0)
            if njobs > 1:
                cp[1] = fire(1)
            for t in range(njobs):
                cp[t].wait()
                pltpu.sync_copy(mv[t % 2], acc.at[idx_v.at[t]], add=True)
                if t + 2 < njobs:
                    cp[t + 2] = fire(t + 2)

        def body(kk, _):
            b = kk * NW + wid

            @pl.when(b < NBH)
            def _():
                pltpu.sync_copy(dst_hbm.at[pl.ds(b * KI, KI)], idx_v)
                one_phase(b * EB, KI)
            return 0

        lax.fori_loop(0, KMAXH, body, 0)

        @pl.when(wid == NW - 1)
        def _():
            pltpu.sync_copy(dst_hbm.at[pl.ds(NBH * KI, TAIL_ROWS)],
                            idx_v.at[pl.ds(0, TAIL_ROWS)])
            one_phase(NBH * EB, TAIL_ROWS)

        plsc.subcore_barrier()

        @pl.when(s == 0)
        def _():
            pltpu.sync_copy(acc, out_hbm.at[c])

    return k(msg, dst2, zeros)


def _final(parts, x, g_bn, b_bn):
    inv_n = 1.0 / N_NODES

    def body(p0_ref, p1_ref, p2_ref, p3_ref, x_ref, g_ref, b_ref, o_ref):
        out = (p0_ref[0] + p0_ref[1] + p1_ref[0] + p1_ref[1]
               + p2_ref[0] + p2_ref[1] + p3_ref[0] + p3_ref[1])
        mean = jnp.sum(out, axis=0, keepdims=True) * inv_n
        d = out - mean
        var = jnp.sum(d * d, axis=0, keepdims=True) * inv_n
        scale = g_ref[...] * lax.rsqrt(var + _BN_EPS)
        o_ref[...] = jnp.maximum(x_ref[...] + d * scale + b_ref[...], 0.0)

    return pl.pallas_call(
        body,
        out_shape=jax.ShapeDtypeStruct((N_NODES, F), jnp.float32),
    )(parts[0], parts[1], parts[2], parts[3], x, g_bn, b_bn)


def kernel(x, edge_index, edge_attr, W1f, b1f, W2f, b2f, W1, b1, W2, b2,
           g_bn, b_bn, g_bni, b_bni):
    src = edge_index[0]
    dst = edge_index[1]
    dst2 = [dst[q * ECH:(q + 1) * ECH].reshape(ECH // IB, IB)
            for q in range(NCHUNK)]
    src2 = [src[q * ECH:(q + 1) * ECH].reshape(ECH // IB, IB)
            for q in range(NCHUNK)]

    wc1 = jnp.concatenate([W1f, W1], axis=1)
    bc1 = jnp.concatenate([b1f, b1]).reshape(1, 2 * F)
    zeros = jnp.zeros((N_NODES, F), jnp.float32)

    g = [_sc_gather(x, dst2[q], src2[q]) for q in range(NCHUNK)]
    mlp = [_edge_mlp(g[q][0], g[q][1], edge_attr, wc1, bc1,
                     W2f, b2f.reshape(1, F), W2, b2.reshape(1, F), half=q)
           for q in range(NCHUNK)]
    moms = [m[2] for m in mlp]
    g1 = g_bni.reshape(1, F)
    b1r = b_bni.reshape(1, F)
    msgs = [_message(mlp[q][0], mlp[q][1], moms, g1, b1r)
            for q in range(NCHUNK)]
    parts = [_sc_scatter(msgs[q], dst2[q], zeros) for q in range(NCHUNK)]
    return _final(parts, x, g_bn.reshape(1, F), b_bn.reshape(1, F))


# speedup vs baseline: 1.0766x; 1.0716x over previous
import functools

import jax
import jax.numpy as jnp
from jax import lax
from jax.experimental import pallas as pl
from jax.experimental.pallas import tpu as pltpu
from jax.experimental.pallas import tpu_sc as plsc

F = 128
N_NODES = 10000
N_EDGES = 320000
NCHUNK = 4
ECH = N_EDGES // NCHUNK

NC = 2
NS = 16
NW = NC * NS

IB = 128
KI = 8
EB = IB * KI
HB = EB // 2
NBH = ECH // EB
KMAXH = -(-NBH // NW)
TAIL_ROWS = (ECH - NBH * EB) // IB
TAIL = TAIL_ROWS * IB

BE = 4000
NEBH = ECH // BE

_BN_EPS = 1e-5


def _sc_mesh():
    return plsc.VectorSubcoreMesh(core_axis_name="c", subcore_axis_name="s",
                                  num_cores=NC, num_subcores=NS)


def _sc_gather(x, dst2, src2):
    @functools.partial(
        pl.kernel,
        out_type=(jax.ShapeDtypeStruct((ECH, F), jnp.float32),
                  jax.ShapeDtypeStruct((ECH, F), jnp.float32)),
        mesh=_sc_mesh(),
        scratch_types=[
            pltpu.VMEM((KI, IB), jnp.int32),
            pltpu.VMEM((2 * IB, F), jnp.float32),
            pltpu.VMEM((2 * IB, F), jnp.float32),
            pltpu.VMEM((2 * IB, F), jnp.float32),
            pltpu.SemaphoreType.DMA,
            pltpu.SemaphoreType.DMA,
            pltpu.SemaphoreType.DMA,
            pltpu.SemaphoreType.DMA,
            pltpu.SemaphoreType.DMA,
            pltpu.SemaphoreType.DMA,
        ],
    )
    def k(x_hbm, dst_hbm, src_hbm, gi_hbm, gj_hbm, idx_v, r0, r1, r2,
          g0, g1, g2, w0, w1, w2):
        wid = lax.axis_index("s") * NC + lax.axis_index("c")
        rows = (r0, r1, r2)
        gsem = (g0, g1, g2)
        wsem = (w0, w1, w2)

        def one_phase(b, idx_hbm, out_hbm, njobs=KI // 2):
            pltpu.sync_copy(idx_hbm.at[pl.ds(b * KI, KI)], idx_v)

            def fire(t):
                i = t % 3
                return [
                    pltpu.async_copy(x_hbm.at[idx_v.at[2 * t + r]],
                                     rows[i].at[pl.ds(r * IB, IB)], gsem[i])
                    for r in range(2)
                ]

            gp = [None] * njobs
            wp = [None] * njobs
            for t in range(min(3, njobs)):
                gp[t] = fire(t)
            for t in range(njobs):
                i = t % 3
                for cp in gp[t]:
                    cp.wait()
                wp[t] = pltpu.async_copy(
                    rows[i], out_hbm.at[pl.ds(b * EB + t * 2 * IB, 2 * IB)],
                    wsem[i])
                if t + 3 < njobs:
                    wp[t].wait()
                    gp[t + 3] = fire(t + 3)
            for t in range(max(0, njobs - 3), njobs):
                wp[t].wait()

        def body(kk, _):
            b = kk * NW + wid

            @pl.when(b < NBH)
            def _():
                one_phase(b, dst_hbm, gi_hbm)
                one_phase(b, src_hbm, gj_hbm)
            return 0

        lax.fori_loop(0, KMAXH, body, 0)

        @pl.when(wid == NW - 1)
        def _():
            for idx_hbm, out_hbm in ((dst_hbm, gi_hbm), (src_hbm, gj_hbm)):
                pltpu.sync_copy(idx_hbm.at[pl.ds(NBH * KI, TAIL_ROWS)],
                                idx_v.at[pl.ds(0, TAIL_ROWS)])
                cps = [
                    pltpu.async_copy(x_hbm.at[idx_v.at[r]],
                                     r0.at[pl.ds(r * IB, IB)], g0)
                    for r in range(TAIL_ROWS)
                ]
                for cp in cps:
                    cp.wait()
                pltpu.sync_copy(r0.at[pl.ds(0, TAIL)],
                                out_hbm.at[pl.ds(NBH * EB, TAIL)])

    return k(x, dst2, src2)


def _edge_mlp(gi, gj, ea, wc1, bc1, w2f, b2f, w2, b2, half):
    def body(gi_ref, gj_ref, ea_ref, wc1_ref, bc1_ref, w2f_ref, b2f_ref,
             w2_ref, b2_ref, hf_ref, h_ref, mom_ref):
        z = jnp.concatenate(
            [gi_ref[...].astype(jnp.bfloat16),
             gj_ref[...].astype(jnp.bfloat16),
             ea_ref[...].astype(jnp.bfloat16)], axis=1)
        u = jnp.dot(z, wc1_ref[...].astype(jnp.bfloat16),
                    preferred_element_type=jnp.float32)
        u = u + bc1_ref[...]
        s = (u * jax.nn.sigmoid(u)).astype(jnp.bfloat16)
        hf = jnp.dot(s[:, :F], w2f_ref[...].astype(jnp.bfloat16),
                     preferred_element_type=jnp.float32) + b2f_ref[...]
        h = jnp.dot(s[:, F:], w2_ref[...].astype(jnp.bfloat16),
                    preferred_element_type=jnp.float32) + b2_ref[...]
        hf_ref[...] = hf.astype(jnp.bfloat16)
        h_ref[...] = h.astype(jnp.bfloat16)

        @pl.when(pl.program_id(0) == 0)
        def _():
            mom_ref[...] = jnp.zeros_like(mom_ref)

        pad = jnp.zeros((6, F), jnp.float32)
        mom_ref[...] += jnp.concatenate(
            [jnp.sum(hf, axis=0, keepdims=True),
             jnp.sum(hf * hf, axis=0, keepdims=True), pad], axis=0)

    eb = lambda i: (i, 0)
    ebo = lambda i: (i + half * NEBH, 0)
    c0 = lambda i: (0, 0)
    return pl.pallas_call(
        body,
        grid=(NEBH,),
        in_specs=[
            pl.BlockSpec((BE, F), eb),
            pl.BlockSpec((BE, F), eb),
            pl.BlockSpec((BE, F), ebo),
            pl.BlockSpec((3 * F, 2 * F), c0),
            pl.BlockSpec((1, 2 * F), c0),
            pl.BlockSpec((F, F), c0),
            pl.BlockSpec((1, F), c0),
            pl.BlockSpec((F, F), c0),
            pl.BlockSpec((1, F), c0),
        ],
        out_specs=[pl.BlockSpec((BE, F), eb),
                   pl.BlockSpec((BE, F), eb),
                   pl.BlockSpec((8, F), c0)],
        out_shape=[jax.ShapeDtypeStruct((ECH, F), jnp.bfloat16),
                   jax.ShapeDtypeStruct((ECH, F), jnp.bfloat16),
                   jax.ShapeDtypeStruct((8, F), jnp.float32)],
    )(gi, gj, ea, wc1, bc1, w2f, b2f, w2, b2)


def _message(hf, h, moms, g_bni, b_bni):
    inv_e = 1.0 / N_EDGES

    def body(hf_ref, h_ref, m0_ref, m1_ref, m2_ref, m3_ref, g_ref, b_ref,
             msg_ref):
        mom = m0_ref[...] + m1_ref[...] + m2_ref[...] + m3_ref[...]
        mean = mom[0:1, :] * inv_e
        var = mom[1:2, :] * inv_e - mean * mean
        scale = g_ref[...] * lax.rsqrt(var + _BN_EPS)
        shift = b_ref[...] - mean * scale
        score = jax.nn.sigmoid(hf_ref[...].astype(jnp.float32) * scale + shift)
        msg_ref[...] = score * h_ref[...].astype(jnp.float32)

    eb = lambda i: (i, 0)
    c0 = lambda i: (0, 0)
    return pl.pallas_call(
        body,
        grid=(NEBH,),
        in_specs=[
            pl.BlockSpec((BE, F), eb),
            pl.BlockSpec((BE, F), eb),
            pl.BlockSpec((8, F), c0),
            pl.BlockSpec((8, F), c0),
            pl.BlockSpec((8, F), c0),
            pl.BlockSpec((8, F), c0),
            pl.BlockSpec((1, F), c0),
            pl.BlockSpec((1, F), c0),
        ],
        out_specs=pl.BlockSpec((BE, F), eb),
        out_shape=jax.ShapeDtypeStruct((ECH, F), jnp.float32),
    )(hf, h, moms[0], moms[1], moms[2], moms[3], g_bni, b_bni)


def _sc_scatter(msg, dst2, zeros):
    @functools.partial(
        pl.kernel,
        out_type=jax.ShapeDtypeStruct((NC, N_NODES, F), jnp.float32),
        mesh=_sc_mesh(),
        scratch_types=[
            pltpu.VMEM((KI, IB), jnp.int32),
            pltpu.VMEM((IB, F), jnp.float32),
            pltpu.VMEM((IB, F), jnp.float32),
            pltpu.SemaphoreType.DMA,
            pltpu.SemaphoreType.DMA,
            pltpu.VMEM_SHARED((N_NODES, F), jnp.float32),
        ],
    )
    def k(msg_hbm, dst_hbm, z_hbm, out_hbm, idx_v, m0, m1, s0, s1, acc):
        c = lax.axis_index("c")
        s = lax.axis_index("s")
        wid = s * NC + c
        mv = (m0, m1)
        msem = (s0, s1)

        @pl.when(s == 0)
        def _():
            pltpu.sync_copy(z_hbm, acc)

        plsc.subcore_barrier()

        def one_phase(e0, njobs):
            def fire(t):
                i = t % 2
                return pltpu.async_copy(
                    msg_hbm.at[pl.ds(e0 + t * IB, IB)], mv[i], msem[i])

            cp = [None] * njobs
            cp[0] = fire(0)
            if njobs > 1:
                cp[1] = fire(1)
            for t in range(njobs):
                cp[t].wait()
                pltpu.sync_copy(mv[t % 2], acc.at[idx_v.at[t]], add=True)
                if t + 2 < njobs:
                    cp[t + 2] = fire(t + 2)

        def body(kk, _):
            b = kk * NW + wid

            @pl.when(b < NBH)
            def _():
                pltpu.sync_copy(dst_hbm.at[pl.ds(b * KI, KI)], idx_v)
                one_phase(b * EB, KI)
            return 0

        lax.fori_loop(0, KMAXH, body, 0)

        @pl.when(wid == NW - 1)
        def _():
            pltpu.sync_copy(dst_hbm.at[pl.ds(NBH * KI, TAIL_ROWS)],
                            idx_v.at[pl.ds(0, TAIL_ROWS)])
            one_phase(NBH * EB, TAIL_ROWS)

        plsc.subcore_barrier()

        @pl.when(s == 0)
        def _():
            pltpu.sync_copy(acc, out_hbm.at[c])

    return k(msg, dst2, zeros)


def _final(parts, x, g_bn, b_bn):
    inv_n = 1.0 / N_NODES

    def body(p0_ref, p1_ref, p2_ref, p3_ref, x_ref, g_ref, b_ref, o_ref):
        out = (p0_ref[0] + p0_ref[1] + p1_ref[0] + p1_ref[1]
               + p2_ref[0] + p2_ref[1] + p3_ref[0] + p3_ref[1])
        mean = jnp.sum(out, axis=0, keepdims=True) * inv_n
        d = out - mean
        var = jnp.sum(d * d, axis=0, keepdims=True) * inv_n
        scale = g_ref[...] * lax.rsqrt(var + _BN_EPS)
        o_ref[...] = jnp.maximum(x_ref[...] + d * scale + b_ref[...], 0.0)

    return pl.pallas_call(
        body,
        out_shape=jax.ShapeDtypeStruct((N_NODES, F), jnp.float32),
    )(parts[0], parts[1], parts[2], parts[3], x, g_bn, b_bn)


def kernel(x, edge_index, edge_attr, W1f, b1f, W2f, b2f, W1, b1, W2, b2,
           g_bn, b_bn, g_bni, b_bni):
    src = edge_index[0]
    dst = edge_index[1]
    dst2 = [dst[q * ECH:(q + 1) * ECH].reshape(ECH // IB, IB)
            for q in range(NCHUNK)]
    src2 = [src[q * ECH:(q + 1) * ECH].reshape(ECH // IB, IB)
            for q in range(NCHUNK)]

    wc1 = jnp.concatenate([W1f, W1], axis=1)
    bc1 = jnp.concatenate([b1f, b1]).reshape(1, 2 * F)
    zeros = jnp.zeros((N_NODES, F), jnp.float32)

    g = [_sc_gather(x, dst2[q], src2[q]) for q in range(NCHUNK)]
    mlp = [_edge_mlp(g[q][0], g[q][1], edge_attr, wc1, bc1,
                     W2f, b2f.reshape(1, F), W2, b2.reshape(1, F), half=q)
           for q in range(NCHUNK)]
    moms = [m[2] for m in mlp]
    g1 = g_bni.reshape(1, F)
    b1r = b_bni.reshape(1, F)
    msgs = [_message(mlp[q][0], mlp[q][1], moms, g1, b1r)
            for q in range(NCHUNK)]
    parts = [_sc_scatter(msgs[q], dst2[q], zeros) for q in range(NCHUNK)]
    return _final(parts, x, g_bn.reshape(1, F), b_bn.reshape(1, F))
